# baseline (speedup 1.0000x reference)
import functools

import jax
import jax.numpy as jnp
from jax import lax
from jax.experimental import pallas as pl
from jax.experimental.pallas import tpu as pltpu
from jax.experimental.pallas import tpu_sc as plsc

MAX_DIST = 10.0
BOX = 21
NV = BOX * BOX * BOX
F = 32

NC = 2
NS = 16
NW = NC * NS
L = 16

SB = 512
NB = SB // 128

NV_PAD = 9344
TRASH = 512
NROWS = NV_PAD + TRASH
ZROWS = NROWS // NS
OROWS = NV_PAD // NS


def _sc_scatter_kernel(n_points: int):
    nsb = -(-n_points // SB)
    j_iters = -(-nsb // NW)
    clamp = n_points - SB

    mesh = plsc.VectorSubcoreMesh(core_axis_name="c", subcore_axis_name="s")

    @functools.partial(
        pl.kernel,
        out_type=jax.ShapeDtypeStruct((NC, NV_PAD, F), jnp.float32),
        mesh=mesh,
        compiler_params=pltpu.CompilerParams(
            needs_layout_passes=False, use_tc_tiling_on_sc=False
        ),
        scratch_types=[
            pltpu.VMEM((3 * SB,), jnp.float32),
            pltpu.VMEM((SB, F), jnp.float32),
            pltpu.VMEM((128,), jnp.int32),
            pltpu.VMEM((128,), jnp.int32),
            pltpu.VMEM((128,), jnp.int32),
            pltpu.VMEM((128,), jnp.int32),
            pltpu.VMEM((ZROWS, F), jnp.float32),
            pltpu.VMEM_SHARED((NROWS, F), jnp.float32),
        ],
    )
    def k(coords_hbm, feats_hbm, out_hbm, cbuf, fbuf, i0, i1, i2, i3, zb, acc):
        c = lax.axis_index("c")
        s = lax.axis_index("s")
        w = s * NC + c
        ibufs = [i0, i1, i2, i3]

        def _zrow(r, carry):
            zb[r, pl.ds(0, L)] = jnp.zeros((L,), jnp.float32)
            zb[r, pl.ds(L, L)] = jnp.zeros((L,), jnp.float32)
            return carry
        lax.fori_loop(0, ZROWS, _zrow, None)
        pltpu.sync_copy(zb, acc.at[pl.ds(s * ZROWS, ZROWS)])
        plsc.subcore_barrier()

        def _body(j, carry):
            sb = w + NW * j

            @pl.when(sb < nsb)
            def _():
                p0 = sb * SB
                p0c = jnp.minimum(p0, clamp)
                shift = p0 - p0c
                pltpu.sync_copy(coords_hbm.at[pl.ds(p0c * 3, 3 * SB)], cbuf)
                pltpu.sync_copy(feats_hbm.at[pl.ds(p0c, SB), :], fbuf)

                lanes = lax.iota(jnp.int32, L)
                for v in range(SB // L):
                    pos = lanes + (v * L)
                    base3 = pos * 3
                    x = plsc.load_gather(cbuf, [base3])
                    y = plsc.load_gather(cbuf, [base3 + 1])
                    z = plsc.load_gather(cbuf, [base3 + 2])
                    tx = x + MAX_DIST
                    ty = y + MAX_DIST
                    tz = z + MAX_DIST
                    ok = (
                        (tx >= -0.5) & (tx <= 20.5)
                        & (ty >= -0.5) & (ty <= 20.5)
                        & (tz >= -0.5) & (tz <= 20.5)
                        & (pos >= shift)
                    )
                    gx = jnp.minimum((tx + 0.5).astype(jnp.int32), BOX - 1)
                    gy = jnp.minimum((ty + 0.5).astype(jnp.int32), BOX - 1)
                    gz = jnp.minimum((tz + 0.5).astype(jnp.int32), BOX - 1)
                    flat = (gx * (BOX * BOX) + gy * BOX) + gz
                    idxv = jnp.where(ok, flat, NV_PAD + pos)
                    ibufs[v // 8][pl.ds((v % 8) * L, L)] = idxv

                for b in range(NB):
                    pltpu.sync_copy(
                        fbuf.at[pl.ds(b * 128, 128), :],
                        acc.at[ibufs[b]],
                        add=True,
                    )
            return carry

        lax.fori_loop(0, j_iters, _body, None)
        plsc.subcore_barrier()

        pltpu.sync_copy(acc.at[pl.ds(s * OROWS, OROWS)], zb.at[pl.ds(0, OROWS)])
        pltpu.sync_copy(zb.at[pl.ds(0, OROWS)], out_hbm.at[c, pl.ds(s * OROWS, OROWS), :])

    return k


def _combine(a_ref, b_ref, o_ref):
    o_ref[...] = a_ref[...] + b_ref[...]


def kernel(coords, features):
    n = coords.shape[1]
    feats = features.reshape(n, F)
    cflat = coords.reshape(3 * n)
    partial = _sc_scatter_kernel(n)(cflat, feats)
    grid = pl.pallas_call(
        _combine,
        out_shape=jax.ShapeDtypeStruct((NV, F), jnp.float32),
    )(partial[0, :NV], partial[1, :NV])
    return grid.reshape(1, BOX, BOX, BOX, F)


# speedup vs baseline: 3.1907x; 3.1907x over previous
import functools

import jax
import jax.numpy as jnp
from jax import lax
from jax.experimental import pallas as pl
from jax.experimental.pallas import tpu as pltpu
from jax.experimental.pallas import tpu_sc as plsc

MAX_DIST = 10.0
BOX = 21
NV = BOX * BOX * BOX
F = 32

NC = 2
NS = 16
NW = NC * NS
L = 16

CH = 128
CPW = 244
NSUP = 61
CPS = CPW // NSUP
N_MAIN = NW * CPW * CH

NV_PAD = 9344
TRASH = 128
NROWS = NV_PAD + TRASH
ZROWS = NROWS // NS
OROWS = NV_PAD // NS


def _sc_scatter_kernel():
    mesh = plsc.VectorSubcoreMesh(core_axis_name="c", subcore_axis_name="s")

    @functools.partial(
        pl.kernel,
        out_type=jax.ShapeDtypeStruct((NC, NV_PAD, 128), jnp.float32),
        mesh=mesh,
        compiler_params=pltpu.CompilerParams(needs_layout_passes=False),
        scratch_types=[
            pltpu.VMEM((CPS * CH,), jnp.float32),
            pltpu.VMEM((CPS * CH,), jnp.float32),
            pltpu.VMEM((CPS * CH,), jnp.float32),
            pltpu.VMEM((CPS, CH), jnp.int32),
            pltpu.VMEM((F, CH), jnp.float32),
            pltpu.VMEM((CH, 128), jnp.float32),
            pltpu.VMEM((80, 128), jnp.float32),
            pltpu.VMEM_SHARED((NROWS, 128), jnp.float32),
        ],
    )
    def k(xs, ys, zs, feats, out_hbm, xb, yb, zb, ib, tb, pb, ob, acc):
        c = lax.axis_index("c")
        s = lax.axis_index("s")
        w = s * NC + c

        zeros = jnp.zeros((L,), jnp.float32)

        def _zrow(r, carry):
            for v in range(128 // L):
                ob[r, pl.ds(v * L, L)] = zeros
            return carry
        lax.fori_loop(0, 80, _zrow, None)
        for kk in range(7):
            pltpu.sync_copy(ob, acc.at[pl.ds(s * ZROWS + kk * 80, 80)])
        pltpu.sync_copy(ob.at[pl.ds(0, 32)], acc.at[pl.ds(s * ZROWS + 560, 32)])

        def _prow(r, carry):
            for v in range(F // L, 128 // L):
                pb[r, pl.ds(v * L, L)] = zeros
            return carry
        lax.fori_loop(0, CH, _prow, None)
        plsc.subcore_barrier()

        lanes = lax.iota(jnp.int32, L)
        base = w * (CPW * CH)

        def _stage(sj, carry):
            p0 = pl.multiple_of(base + sj * (CPS * CH), CPS * CH)
            pltpu.sync_copy(xs.at[pl.ds(p0, CPS * CH)], xb)
            pltpu.sync_copy(ys.at[pl.ds(p0, CPS * CH)], yb)
            pltpu.sync_copy(zs.at[pl.ds(p0, CPS * CH)], zb)

            def _quant(cc, carry):
                for v in range(CH // L):
                    pos = cc * CH + v * L
                    tx = xb[pl.ds(pos, L)] + MAX_DIST
                    ty = yb[pl.ds(pos, L)] + MAX_DIST
                    tz = zb[pl.ds(pos, L)] + MAX_DIST
                    ok = (
                        (tx >= -0.5) & (tx <= 20.5)
                        & (ty >= -0.5) & (ty <= 20.5)
                        & (tz >= -0.5) & (tz <= 20.5)
                    )
                    gx = jnp.minimum((tx + 0.5).astype(jnp.int32), BOX - 1)
                    gy = jnp.minimum((ty + 0.5).astype(jnp.int32), BOX - 1)
                    gz = jnp.minimum((tz + 0.5).astype(jnp.int32), BOX - 1)
                    flat = (gx * (BOX * BOX) + gy * BOX) + gz
                    trash = NV_PAD + (v * L) + lanes
                    ib[cc, pl.ds(v * L, L)] = jnp.where(ok, flat, trash)
                return carry
            lax.fori_loop(0, CPS, _quant, None)

            def _chunk(cc, carry):
                pc = pl.multiple_of(p0 + cc * CH, CH)
                pltpu.sync_copy(feats.at[:, pl.ds(pc, CH)], tb)

                def _xpose(pp, carry):
                    for u in range(8):
                        p = pp * 8 + u
                        pvec = jnp.full((L,), p, jnp.int32)
                        lo = plsc.load_gather(tb, [lanes, pvec])
                        hi = plsc.load_gather(tb, [lanes + L, pvec])
                        pb[p, pl.ds(0, L)] = lo
                        pb[p, pl.ds(L, L)] = hi
                    return carry
                lax.fori_loop(0, CH // 8, _xpose, None)
                pltpu.sync_copy(pb, acc.at[ib.at[cc]], add=True)
                return carry
            lax.fori_loop(0, CPS, _chunk, None)
            return carry

        lax.fori_loop(0, NSUP, _stage, None)

        plsc.subcore_barrier()

        for kk in range(7):
            pltpu.sync_copy(acc.at[pl.ds(s * OROWS + kk * 80, 80)], ob)
            pltpu.sync_copy(ob, out_hbm.at[c, pl.ds(s * OROWS + kk * 80, 80), :])
        pltpu.sync_copy(acc.at[pl.ds(s * OROWS + 560, 24)], ob.at[pl.ds(0, 24)])
        pltpu.sync_copy(ob.at[pl.ds(0, 24)], out_hbm.at[c, pl.ds(s * OROWS + 560, 24), :])

    return k


def _combine(p_ref, tx_ref, ty_ref, tz_ref, tf_ref, o_ref):
    g = p_ref[0, :NV, :F] + p_ref[1, :NV, :F]
    tx = tx_ref[...] + MAX_DIST
    ty = ty_ref[...] + MAX_DIST
    tz = tz_ref[...] + MAX_DIST
    ok = (
        (tx >= -0.5) & (tx <= 20.5)
        & (ty >= -0.5) & (ty <= 20.5)
        & (tz >= -0.5) & (tz <= 20.5)
    )
    gx = jnp.minimum((tx + 0.5).astype(jnp.int32), BOX - 1)
    gy = jnp.minimum((ty + 0.5).astype(jnp.int32), BOX - 1)
    gz = jnp.minimum((tz + 0.5).astype(jnp.int32), BOX - 1)
    flat = jnp.where(ok, (gx * (BOX * BOX) + gy * BOX) + gz, -1)
    onehot = (
        lax.broadcasted_iota(jnp.int32, (NV, flat.shape[0]), 0) == flat[None, :]
    ).astype(jnp.float32)
    o_ref[...] = g + jnp.dot(onehot, tf_ref[...],
                             preferred_element_type=jnp.float32)


def kernel(coords, features):
    n = coords.shape[1]
    featT = jnp.swapaxes(features, 1, 2).reshape(F, n)
    xs = coords[0, :, 0]
    ys = coords[0, :, 1]
    zs = coords[0, :, 2]
    partial = _sc_scatter_kernel()(xs, ys, zs, featT)
    grid = pl.pallas_call(
        _combine,
        out_shape=jax.ShapeDtypeStruct((NV, F), jnp.float32),
    )(
        partial,
        coords[0, N_MAIN:, 0],
        coords[0, N_MAIN:, 1],
        coords[0, N_MAIN:, 2],
        features[0, N_MAIN:, :],
    )
    return grid.reshape(1, BOX, BOX, BOX, F)


# speedup vs baseline: 3.4996x; 1.0968x over previous
import functools

import jax
import jax.numpy as jnp
from jax import lax
from jax.experimental import pallas as pl
from jax.experimental.pallas import tpu as pltpu
from jax.experimental.pallas import tpu_sc as plsc

MAX_DIST = 10.0
BOX = 21
NV = BOX * BOX * BOX
F = 32

NC = 2
NS = 16
NW = NC * NS
L = 16

CH = 128
CPW = 244
NSUP = 61
CPS = CPW // NSUP
N_MAIN = NW * CPW * CH

NV_PAD = 9344
TRASH = 128
NROWS = NV_PAD + TRASH
ZROWS = NROWS // NS
OROWS = NV_PAD // NS


def _sc_scatter_kernel():
    mesh = plsc.VectorSubcoreMesh(core_axis_name="c", subcore_axis_name="s")

    @functools.partial(
        pl.kernel,
        out_type=jax.ShapeDtypeStruct((NC, NV_PAD, F), jnp.float32),
        mesh=mesh,
        compiler_params=pltpu.CompilerParams(needs_layout_passes=False),
        scratch_types=[
            pltpu.VMEM((CPS * CH,), jnp.float32),
            pltpu.VMEM((CPS * CH,), jnp.float32),
            pltpu.VMEM((CPS * CH,), jnp.float32),
            pltpu.VMEM((CPS, CH), jnp.int32),
            pltpu.VMEM((F, CH), jnp.float32),
            pltpu.VMEM((CH, F), jnp.float32),
            pltpu.VMEM((80, F), jnp.float32),
            pltpu.VMEM_SHARED((NROWS, F), jnp.float32),
        ],
    )
    def k(xs, ys, zs, feats, out_hbm, xb, yb, zb, ib, tb, pb, ob, acc):
        c = lax.axis_index("c")
        s = lax.axis_index("s")
        w = s * NC + c

        zeros = jnp.zeros((L,), jnp.float32)

        def _zrow(r, carry):
            for v in range(F // L):
                ob[r, pl.ds(v * L, L)] = zeros
            return carry
        lax.fori_loop(0, 80, _zrow, None)
        for kk in range(7):
            pltpu.sync_copy(ob, acc.at[pl.ds(s * ZROWS + kk * 80, 80)])
        pltpu.sync_copy(ob.at[pl.ds(0, 32)], acc.at[pl.ds(s * ZROWS + 560, 32)])

        plsc.subcore_barrier()

        lanes = lax.iota(jnp.int32, L)
        base = w * (CPW * CH)

        def _stage(sj, carry):
            p0 = pl.multiple_of(base + sj * (CPS * CH), CPS * CH)
            pltpu.sync_copy(xs.at[pl.ds(p0, CPS * CH)], xb)
            pltpu.sync_copy(ys.at[pl.ds(p0, CPS * CH)], yb)
            pltpu.sync_copy(zs.at[pl.ds(p0, CPS * CH)], zb)

            def _quant(cc, carry):
                for v in range(CH // L):
                    pos = cc * CH + v * L
                    tx = xb[pl.ds(pos, L)] + MAX_DIST
                    ty = yb[pl.ds(pos, L)] + MAX_DIST
                    tz = zb[pl.ds(pos, L)] + MAX_DIST
                    ok = (
                        (tx >= -0.5) & (tx <= 20.5)
                        & (ty >= -0.5) & (ty <= 20.5)
                        & (tz >= -0.5) & (tz <= 20.5)
                    )
                    gx = jnp.minimum((tx + 0.5).astype(jnp.int32), BOX - 1)
                    gy = jnp.minimum((ty + 0.5).astype(jnp.int32), BOX - 1)
                    gz = jnp.minimum((tz + 0.5).astype(jnp.int32), BOX - 1)
                    flat = (gx * (BOX * BOX) + gy * BOX) + gz
                    trash = NV_PAD + (v * L) + lanes
                    ib[cc, pl.ds(v * L, L)] = jnp.where(ok, flat, trash)
                return carry
            lax.fori_loop(0, CPS, _quant, None)

            def _chunk(cc, carry):
                pc = pl.multiple_of(p0 + cc * CH, CH)
                pltpu.sync_copy(feats.at[:, pl.ds(pc, CH)], tb)

                def _xpose(pp, carry):
                    for u in range(8):
                        p = pp * 8 + u
                        pvec = jnp.full((L,), p, jnp.int32)
                        lo = plsc.load_gather(tb, [lanes, pvec])
                        hi = plsc.load_gather(tb, [lanes + L, pvec])
                        pb[p, pl.ds(0, L)] = lo
                        pb[p, pl.ds(L, L)] = hi
                    return carry
                lax.fori_loop(0, CH // 8, _xpose, None)
                pltpu.sync_copy(pb, acc.at[ib.at[cc]], add=True)
                return carry
            lax.fori_loop(0, CPS, _chunk, None)
            return carry

        lax.fori_loop(0, NSUP, _stage, None)

        plsc.subcore_barrier()

        for kk in range(7):
            pltpu.sync_copy(acc.at[pl.ds(s * OROWS + kk * 80, 80)], ob)
            pltpu.sync_copy(ob, out_hbm.at[c, pl.ds(s * OROWS + kk * 80, 80), :])
        pltpu.sync_copy(acc.at[pl.ds(s * OROWS + 560, 24)], ob.at[pl.ds(0, 24)])
        pltpu.sync_copy(ob.at[pl.ds(0, 24)], out_hbm.at[c, pl.ds(s * OROWS + 560, 24), :])

    return k


def _combine(p_ref, tx_ref, ty_ref, tz_ref, tf_ref, o_ref):
    g = p_ref[0, :NV, :] + p_ref[1, :NV, :]
    tx = tx_ref[...] + MAX_DIST
    ty = ty_ref[...] + MAX_DIST
    tz = tz_ref[...] + MAX_DIST
    ok = (
        (tx >= -0.5) & (tx <= 20.5)
        & (ty >= -0.5) & (ty <= 20.5)
        & (tz >= -0.5) & (tz <= 20.5)
    )
    gx = jnp.minimum((tx + 0.5).astype(jnp.int32), BOX - 1)
    gy = jnp.minimum((ty + 0.5).astype(jnp.int32), BOX - 1)
    gz = jnp.minimum((tz + 0.5).astype(jnp.int32), BOX - 1)
    flat = jnp.where(ok, (gx * (BOX * BOX) + gy * BOX) + gz, -1)
    onehot = (
        lax.broadcasted_iota(jnp.int32, (NV, flat.shape[0]), 0) == flat[None, :]
    ).astype(jnp.float32)
    o_ref[...] = g + jnp.dot(onehot, tf_ref[...],
                             preferred_element_type=jnp.float32)


def kernel(coords, features):
    n = coords.shape[1]
    featT = jnp.swapaxes(features, 1, 2).reshape(F, n)
    xs = coords[0, :, 0]
    ys = coords[0, :, 1]
    zs = coords[0, :, 2]
    partial = _sc_scatter_kernel()(xs, ys, zs, featT)
    grid = pl.pallas_call(
        _combine,
        out_shape=jax.ShapeDtypeStruct((NV, F), jnp.float32),
    )(
        partial,
        coords[0, N_MAIN:, 0],
        coords[0, N_MAIN:, 1],
        coords[0, N_MAIN:, 2],
        features[0, N_MAIN:, :],
    )
    return grid.reshape(1, BOX, BOX, BOX, F)


# speedup vs baseline: 4.0320x; 1.1521x over previous
import functools

import jax
import jax.numpy as jnp
from jax import lax
from jax.experimental import pallas as pl
from jax.experimental.pallas import tpu as pltpu
from jax.experimental.pallas import tpu_sc as plsc

MAX_DIST = 10.0
BOX = 21
NV = BOX * BOX * BOX
F = 32

NC = 2
NS = 16
NW = NC * NS
L = 16

CH = 128
CPW = 244
NSUP = 61
CPS = CPW // NSUP
N_MAIN = NW * CPW * CH

NV_PAD = 9344
TRASH = 128
NROWS = NV_PAD + TRASH
ZROWS = NROWS // NS
OROWS = NV_PAD // NS


def _sc_scatter_kernel():
    mesh = plsc.VectorSubcoreMesh(core_axis_name="c", subcore_axis_name="s")

    @functools.partial(
        pl.kernel,
        out_type=jax.ShapeDtypeStruct((NC, NV_PAD, 128), jnp.float32),
        mesh=mesh,
        compiler_params=pltpu.CompilerParams(needs_layout_passes=False),
        scratch_types=[
            pltpu.VMEM((CPS * CH,), jnp.float32),
            pltpu.VMEM((CPS * CH,), jnp.float32),
            pltpu.VMEM((CPS * CH,), jnp.float32),
            pltpu.VMEM((CPS, CH), jnp.int32),
            pltpu.VMEM((F, CH), jnp.float32),
            pltpu.VMEM((F, CH), jnp.float32),
            pltpu.VMEM((CH, 128), jnp.float32),
            pltpu.VMEM((CH, 128), jnp.float32),
            pltpu.VMEM((80, 128), jnp.float32),
            pltpu.VMEM_SHARED((NROWS, 128), jnp.float32),
            pltpu.SemaphoreType.DMA,
            pltpu.SemaphoreType.DMA,
            pltpu.SemaphoreType.DMA,
            pltpu.SemaphoreType.DMA,
        ],
    )
    def k(xs, ys, zs, feats, out_hbm, xb, yb, zb, ib, tb, tb2, pb, pb2, ob,
          acc, st0, st1, ss0, ss1):
        c = lax.axis_index("c")
        s = lax.axis_index("s")
        w = s * NC + c

        zeros = jnp.zeros((L,), jnp.float32)

        def _zrow(r, carry):
            for v in range(128 // L):
                ob[r, pl.ds(v * L, L)] = zeros
            return carry
        lax.fori_loop(0, 80, _zrow, None)

        def _prow(r, carry):
            for v in range(F // L, 128 // L):
                pb[r, pl.ds(v * L, L)] = zeros
                pb2[r, pl.ds(v * L, L)] = zeros
            return carry
        lax.fori_loop(0, CH, _prow, None)
        for kk in range(7):
            pltpu.sync_copy(ob, acc.at[pl.ds(s * ZROWS + kk * 80, 80)])
        pltpu.sync_copy(ob.at[pl.ds(0, 32)], acc.at[pl.ds(s * ZROWS + 560, 32)])

        plsc.subcore_barrier()

        lanes = lax.iota(jnp.int32, L)
        base = w * (CPW * CH)

        def _stage(sj, carry):
            p0 = pl.multiple_of(base + sj * (CPS * CH), CPS * CH)
            pltpu.sync_copy(xs.at[pl.ds(p0, CPS * CH)], xb)
            pltpu.sync_copy(ys.at[pl.ds(p0, CPS * CH)], yb)
            pltpu.sync_copy(zs.at[pl.ds(p0, CPS * CH)], zb)

            def _quant(cc, carry):
                for v in range(CH // L):
                    pos = cc * CH + v * L
                    tx = xb[pl.ds(pos, L)] + MAX_DIST
                    ty = yb[pl.ds(pos, L)] + MAX_DIST
                    tz = zb[pl.ds(pos, L)] + MAX_DIST
                    ok = (
                        (tx >= -0.5) & (tx <= 20.5)
                        & (ty >= -0.5) & (ty <= 20.5)
                        & (tz >= -0.5) & (tz <= 20.5)
                    )
                    gx = jnp.minimum((tx + 0.5).astype(jnp.int32), BOX - 1)
                    gy = jnp.minimum((ty + 0.5).astype(jnp.int32), BOX - 1)
                    gz = jnp.minimum((tz + 0.5).astype(jnp.int32), BOX - 1)
                    flat = (gx * (BOX * BOX) + gy * BOX) + gz
                    trash = NV_PAD + (v * L) + lanes
                    ib[cc, pl.ds(v * L, L)] = jnp.where(ok, flat, trash)
                return carry
            lax.fori_loop(0, CPS, _quant, None)

            tbs = [tb, tb2]
            pbs = [pb, pb2]
            tsem = [st0, st1]
            ssem = [ss0, ss1]

            def _mk_xpose(tbuf, pbuf):
                def _xpose(pp, carry):
                    for u in range(8):
                        p = pp * 8 + u
                        pvec = jnp.full((L,), p, jnp.int32)
                        lo = plsc.load_gather(tbuf, [lanes, pvec])
                        hi = plsc.load_gather(tbuf, [lanes + L, pvec])
                        pbuf[p, pl.ds(0, L)] = lo
                        pbuf[p, pl.ds(L, L)] = hi
                    return carry
                return _xpose

            def _tile_dma(cc, b):
                pc = pl.multiple_of(p0 + cc * CH, CH)
                return pltpu.async_copy(
                    feats.at[:, pl.ds(pc, CH)], tbs[b], tsem[b]
                )

            d = [_tile_dma(0, 0), _tile_dma(1, 1)]
            sc = [None, None]
            for cc in range(CPS):
                b = cc % 2
                d[b].wait()
                if sc[b] is not None:
                    sc[b].wait()
                lax.fori_loop(0, CH // 8, _mk_xpose(tbs[b], pbs[b]), None)
                sc[b] = pltpu.async_copy(
                    pbs[b], acc.at[ib.at[cc]], ssem[b], add=True
                )
                if cc + 2 < CPS:
                    d[b] = _tile_dma(cc + 2, b)
            sc[0].wait()
            sc[1].wait()
            return carry

        lax.fori_loop(0, NSUP, _stage, None)

        plsc.subcore_barrier()

        for kk in range(7):
            pltpu.sync_copy(acc.at[pl.ds(s * OROWS + kk * 80, 80)], ob)
            pltpu.sync_copy(ob, out_hbm.at[c, pl.ds(s * OROWS + kk * 80, 80), :])
        pltpu.sync_copy(acc.at[pl.ds(s * OROWS + 560, 24)], ob.at[pl.ds(0, 24)])
        pltpu.sync_copy(ob.at[pl.ds(0, 24)], out_hbm.at[c, pl.ds(s * OROWS + 560, 24), :])

    return k


def _combine(p_ref, tx_ref, ty_ref, tz_ref, tf_ref, o_ref):
    g = p_ref[0, :NV, :F] + p_ref[1, :NV, :F]
    tx = tx_ref[...] + MAX_DIST
    ty = ty_ref[...] + MAX_DIST
    tz = tz_ref[...] + MAX_DIST
    ok = (
        (tx >= -0.5) & (tx <= 20.5)
        & (ty >= -0.5) & (ty <= 20.5)
        & (tz >= -0.5) & (tz <= 20.5)
    )
    gx = jnp.minimum((tx + 0.5).astype(jnp.int32), BOX - 1)
    gy = jnp.minimum((ty + 0.5).astype(jnp.int32), BOX - 1)
    gz = jnp.minimum((tz + 0.5).astype(jnp.int32), BOX - 1)
    flat = jnp.where(ok, (gx * (BOX * BOX) + gy * BOX) + gz, -1)
    onehot = (
        lax.broadcasted_iota(jnp.int32, (NV, flat.shape[0]), 0) == flat[None, :]
    ).astype(jnp.float32)
    o_ref[...] = g + jnp.dot(onehot, tf_ref[...],
                             preferred_element_type=jnp.float32)


def kernel(coords, features):
    n = coords.shape[1]
    featT = jnp.swapaxes(features, 1, 2).reshape(F, n)
    xs = coords[0, :, 0]
    ys = coords[0, :, 1]
    zs = coords[0, :, 2]
    partial = _sc_scatter_kernel()(xs, ys, zs, featT)
    grid = pl.pallas_call(
        _combine,
        out_shape=jax.ShapeDtypeStruct((NV, F), jnp.float32),
    )(
        partial,
        coords[0, N_MAIN:, 0],
        coords[0, N_MAIN:, 1],
        coords[0, N_MAIN:, 2],
        features[0, N_MAIN:, :],
    )
    return grid.reshape(1, BOX, BOX, BOX, F)


# speedup vs baseline: 4.1447x; 1.0280x over previous
import functools

import jax
import jax.numpy as jnp
from jax import lax
from jax.experimental import pallas as pl
from jax.experimental.pallas import tpu as pltpu
from jax.experimental.pallas import tpu_sc as plsc

MAX_DIST = 10.0
BOX = 21
NV = BOX * BOX * BOX
F = 32

NC = 2
NS = 16
NW = NC * NS
L = 16

CH = 128
CPW = 244
NSUP = 61
CPS = CPW // NSUP
N_MAIN = NW * CPW * CH

NV_PAD = 9344
TRASH = 128
NROWS = NV_PAD + TRASH
ZROWS = NROWS // NS
OROWS = NV_PAD // NS


def _sc_scatter_kernel():
    mesh = plsc.VectorSubcoreMesh(core_axis_name="c", subcore_axis_name="s")

    @functools.partial(
        pl.kernel,
        out_type=jax.ShapeDtypeStruct((NC, NV_PAD, 128), jnp.float32),
        mesh=mesh,
        compiler_params=pltpu.CompilerParams(needs_layout_passes=False),
        scratch_types=[
            pltpu.VMEM((CPS * CH,), jnp.float32),
            pltpu.VMEM((CPS * CH,), jnp.float32),
            pltpu.VMEM((CPS * CH,), jnp.float32),
            pltpu.VMEM((2 * CPS, CH), jnp.int32),
            pltpu.VMEM((F, CH), jnp.float32),
            pltpu.VMEM((F, CH), jnp.float32),
            pltpu.VMEM((CH, 128), jnp.float32),
            pltpu.VMEM((CH, 128), jnp.float32),
            pltpu.VMEM((80, 128), jnp.float32),
            pltpu.VMEM_SHARED((NROWS, 128), jnp.float32),
            pltpu.SemaphoreType.DMA,
            pltpu.SemaphoreType.DMA,
            pltpu.SemaphoreType.DMA,
            pltpu.SemaphoreType.DMA,
        ],
    )
    def k(xs, ys, zs, feats, out_hbm, xb, yb, zb, ib, tb, tb2, pb, pb2, ob,
          acc, st0, st1, ss0, ss1):
        c = lax.axis_index("c")
        s = lax.axis_index("s")
        w = s * NC + c

        zeros = jnp.zeros((L,), jnp.float32)

        def _zrow(r, carry):
            for v in range(128 // L):
                ob[r, pl.ds(v * L, L)] = zeros
            return carry
        lax.fori_loop(0, 80, _zrow, None)

        def _prow(r, carry):
            for v in range(F // L, 128 // L):
                pb[r, pl.ds(v * L, L)] = zeros
                pb2[r, pl.ds(v * L, L)] = zeros
            return carry
        lax.fori_loop(0, CH, _prow, None)
        for kk in range(7):
            pltpu.sync_copy(ob, acc.at[pl.ds(s * ZROWS + kk * 80, 80)])
        pltpu.sync_copy(ob.at[pl.ds(0, 32)], acc.at[pl.ds(s * ZROWS + 560, 32)])

        plsc.subcore_barrier()

        lanes = lax.iota(jnp.int32, L)
        base = w * (CPW * CH)
        pbs_outer = [pb, pb2]
        ssem_outer = [ss0, ss1]

        def _stage(sj, carry):
            p0 = pl.multiple_of(base + sj * (CPS * CH), CPS * CH)
            pltpu.sync_copy(xs.at[pl.ds(p0, CPS * CH)], xb)
            pltpu.sync_copy(ys.at[pl.ds(p0, CPS * CH)], yb)
            pltpu.sync_copy(zs.at[pl.ds(p0, CPS * CH)], zb)

            parity = (sj % 2) * CPS

            def _quant(cc, carry):
                for v in range(CH // L):
                    pos = cc * CH + v * L
                    tx = xb[pl.ds(pos, L)] + MAX_DIST
                    ty = yb[pl.ds(pos, L)] + MAX_DIST
                    tz = zb[pl.ds(pos, L)] + MAX_DIST
                    ok = (
                        (tx >= -0.5) & (tx <= 20.5)
                        & (ty >= -0.5) & (ty <= 20.5)
                        & (tz >= -0.5) & (tz <= 20.5)
                    )
                    gx = jnp.minimum((tx + 0.5).astype(jnp.int32), BOX - 1)
                    gy = jnp.minimum((ty + 0.5).astype(jnp.int32), BOX - 1)
                    gz = jnp.minimum((tz + 0.5).astype(jnp.int32), BOX - 1)
                    flat = (gx * (BOX * BOX) + gy * BOX) + gz
                    trash = NV_PAD + (v * L) + lanes
                    ib[parity + cc, pl.ds(v * L, L)] = jnp.where(ok, flat, trash)
                return carry
            lax.fori_loop(0, CPS, _quant, None)

            tbs = [tb, tb2]
            pbs = [pb, pb2]
            tsem = [st0, st1]
            ssem = [ss0, ss1]

            def _mk_xpose(tbuf, pbuf):
                def _xpose(pp, carry):
                    for u in range(8):
                        p = pp * 8 + u
                        pvec = jnp.full((L,), p, jnp.int32)
                        lo = plsc.load_gather(tbuf, [lanes, pvec])
                        hi = plsc.load_gather(tbuf, [lanes + L, pvec])
                        pbuf[p, pl.ds(0, L)] = lo
                        pbuf[p, pl.ds(L, L)] = hi
                    return carry
                return _xpose

            def _tile_dma(cc, b):
                pc = pl.multiple_of(p0 + cc * CH, CH)
                return pltpu.async_copy(
                    feats.at[:, pl.ds(pc, CH)], tbs[b], tsem[b]
                )

            d = [_tile_dma(0, 0), _tile_dma(1, 1)]
            for cc in range(CPS):
                b = cc % 2
                d[b].wait()

                def _scat_wait(b=b, cc=cc):
                    pltpu.make_async_copy(
                        pbs[b], acc.at[ib.at[parity + cc]], ssem[b]
                    ).wait()

                if cc < 2:
                    pl.when(sj > 0)(_scat_wait)
                else:
                    _scat_wait()
                lax.fori_loop(0, CH // 8, _mk_xpose(tbs[b], pbs[b]), None)
                pltpu.async_copy(
                    pbs[b], acc.at[ib.at[parity + cc]], ssem[b], add=True
                )
                if cc + 2 < CPS:
                    d[b] = _tile_dma(cc + 2, b)
            return carry

        lax.fori_loop(0, NSUP, _stage, None)
        for b in range(2):
            pltpu.make_async_copy(pbs_outer[b], acc.at[ib.at[b]], ssem_outer[b]).wait()

        plsc.subcore_barrier()

        for kk in range(7):
            pltpu.sync_copy(acc.at[pl.ds(s * OROWS + kk * 80, 80)], ob)
            pltpu.sync_copy(ob, out_hbm.at[c, pl.ds(s * OROWS + kk * 80, 80), :])
        pltpu.sync_copy(acc.at[pl.ds(s * OROWS + 560, 24)], ob.at[pl.ds(0, 24)])
        pltpu.sync_copy(ob.at[pl.ds(0, 24)], out_hbm.at[c, pl.ds(s * OROWS + 560, 24), :])

    return k


def _combine(p_ref, tx_ref, ty_ref, tz_ref, tf_ref, o_ref):
    g = p_ref[0, :NV, :F] + p_ref[1, :NV, :F]
    tx = tx_ref[...] + MAX_DIST
    ty = ty_ref[...] + MAX_DIST
    tz = tz_ref[...] + MAX_DIST
    ok = (
        (tx >= -0.5) & (tx <= 20.5)
        & (ty >= -0.5) & (ty <= 20.5)
        & (tz >= -0.5) & (tz <= 20.5)
    )
    gx = jnp.minimum((tx + 0.5).astype(jnp.int32), BOX - 1)
    gy = jnp.minimum((ty + 0.5).astype(jnp.int32), BOX - 1)
    gz = jnp.minimum((tz + 0.5).astype(jnp.int32), BOX - 1)
    flat = jnp.where(ok, (gx * (BOX * BOX) + gy * BOX) + gz, -1)
    onehot = (
        lax.broadcasted_iota(jnp.int32, (NV, flat.shape[0]), 0) == flat[None, :]
    ).astype(jnp.float32)
    o_ref[...] = g + jnp.dot(onehot, tf_ref[...],
                             preferred_element_type=jnp.float32)


def kernel(coords, features):
    n = coords.shape[1]
    featT = jnp.swapaxes(features, 1, 2).reshape(F, n)
    xs = coords[0, :, 0]
    ys = coords[0, :, 1]
    zs = coords[0, :, 2]
    partial = _sc_scatter_kernel()(xs, ys, zs, featT)
    grid = pl.pallas_call(
        _combine,
        out_shape=jax.ShapeDtypeStruct((NV, F), jnp.float32),
    )(
        partial,
        coords[0, N_MAIN:, 0],
        coords[0, N_MAIN:, 1],
        coords[0, N_MAIN:, 2],
        features[0, N_MAIN:, :],
    )
    return grid.reshape(1, BOX, BOX, BOX, F)
